# baseline (speedup 1.0000x reference)
import jax
import jax.numpy as jnp
from jax.experimental import pallas as pl
from jax.experimental.pallas import tpu as pltpu

_B = 16384
_C = 1000
_BLK = 2048
_NSTEP = _B // _BLK


def _loss_body(lab_ref, pred_hbm, out_ref, buf0, buf1, sem0, sem1):
    bufs, sems = (buf0, buf1), (sem0, sem1)

    def _copy(k):
        return pltpu.make_async_copy(
            pred_hbm.at[pl.ds(k * _BLK, _BLK)], bufs[k % 2], sems[k % 2])

    _copy(0).start()
    cols = jax.lax.broadcasted_iota(jnp.int32, (_BLK, _C), 1)
    ones = jnp.ones((_C, 8), jnp.float32)
    total = 0.0
    for k in range(_NSTEP):
        _copy(k).wait()
        if k + 1 < _NSTEP:
            _copy(k + 1).start()
        x = bufs[k % 2][...]
        labT = lab_ref[k, 0].reshape(_BLK, 1)
        masked = jnp.where(cols == labT, x, 0.0)
        sel = jax.lax.dot_general(masked, ones, (((1,), (0,)), ((), ())),
                                  preferred_element_type=jnp.float32)
        total += jnp.sum(jnp.log(sel + 1e-8)) * 0.125
    out_ref[0, 0] = total * (-1.0 / (_B * _C))


def kernel(pred, label):
    lab3 = label.astype(jnp.int32).reshape(_NSTEP, 1, _BLK)
    out = pl.pallas_call(
        _loss_body,
        in_specs=[
            pl.BlockSpec(memory_space=pltpu.VMEM),
            pl.BlockSpec(memory_space=pltpu.HBM),
        ],
        out_specs=pl.BlockSpec(memory_space=pltpu.SMEM),
        out_shape=jax.ShapeDtypeStruct((1, 1), jnp.float32),
        scratch_shapes=[
            pltpu.VMEM((_BLK, _C), jnp.float32),
            pltpu.VMEM((_BLK, _C), jnp.float32),
            pltpu.SemaphoreType.DMA,
            pltpu.SemaphoreType.DMA,
        ],
    )(lab3, pred)
    return out[0, 0]


# speedup vs baseline: 1.0578x; 1.0578x over previous
import jax
import jax.numpy as jnp
from jax.experimental import pallas as pl
from jax.experimental.pallas import tpu as pltpu

_B = 16384
_C = 1000
_BLK = 1024
_NSPLIT = 4
_NSTEP = _B // (_BLK * _NSPLIT)


def _loss_body(lab_ref, *refs):
    pred_refs = refs[:_NSPLIT]
    out_ref, acc_ref = refs[_NSPLIT], refs[_NSPLIT + 1]
    i = pl.program_id(0)

    @pl.when(i == 0)
    def _():
        acc_ref[0, 0] = 0.0

    cols = jax.lax.broadcasted_iota(jnp.int32, (_BLK, _C), 1)
    ones = jnp.ones((_C, 8), jnp.float32)
    part = 0.0
    for q in range(_NSPLIT):
        x = pred_refs[q][...]
        labT = lab_ref[0, q].reshape(_BLK, 1)
        masked = jnp.where(cols == labT, x, 0.0)
        sel = jax.lax.dot_general(masked, ones, (((1,), (0,)), ((), ())),
                                  preferred_element_type=jnp.float32)
        part += jnp.sum(jnp.log(sel + 1e-8)) * 0.125
    acc_ref[0, 0] += part

    @pl.when(i == _NSTEP - 1)
    def _():
        out_ref[0, 0] = acc_ref[0, 0] * (-1.0 / (_B * _C))


def kernel(pred, label):
    lab3 = label.astype(jnp.int32).reshape(
        _NSPLIT, _NSTEP, _BLK).transpose(1, 0, 2)
    qrows = _B // _NSPLIT // _BLK
    in_specs = [pl.BlockSpec((1, _NSPLIT, _BLK), lambda i: (i, 0, 0))]
    for q in range(_NSPLIT):
        in_specs.append(
            pl.BlockSpec((_BLK, _C), lambda i, q=q: (q * qrows + i, 0)))
    out = pl.pallas_call(
        _loss_body,
        grid=(_NSTEP,),
        in_specs=in_specs,
        out_specs=pl.BlockSpec(memory_space=pltpu.SMEM),
        out_shape=jax.ShapeDtypeStruct((1, 1), jnp.float32),
        scratch_shapes=[pltpu.SMEM((1, 1), jnp.float32)],
    )(lab3, pred, pred, pred, pred)
    return out[0, 0]


# speedup vs baseline: 1.1049x; 1.0445x over previous
import jax
import jax.numpy as jnp
from jax.experimental import pallas as pl
from jax.experimental.pallas import tpu as pltpu

_B = 16384
_C = 1000
_BLK = 2048
_NSPLIT = 1
_NSTEP = _B // (_BLK * _NSPLIT)


def _loss_body(lab_ref, *refs):
    pred_refs = refs[:_NSPLIT]
    out_ref, acc_ref = refs[_NSPLIT], refs[_NSPLIT + 1]
    i = pl.program_id(0)

    @pl.when(i == 0)
    def _():
        acc_ref[0, 0] = 0.0

    cols = jax.lax.broadcasted_iota(jnp.int32, (_BLK, _C), 1)
    ones = jnp.ones((_C, 8), jnp.float32)
    part = 0.0
    for q in range(_NSPLIT):
        x = pred_refs[q][...]
        labT = lab_ref[0, q].reshape(_BLK, 1)
        masked = jnp.where(cols == labT, x, 0.0)
        sel = jax.lax.dot_general(masked, ones, (((1,), (0,)), ((), ())),
                                  preferred_element_type=jnp.float32)
        part += jnp.sum(jnp.log(sel + 1e-8)) * 0.125
    acc_ref[0, 0] += part

    @pl.when(i == _NSTEP - 1)
    def _():
        out_ref[0, 0] = acc_ref[0, 0] * (-1.0 / (_B * _C))


def kernel(pred, label):
    lab3 = label.astype(jnp.int32).reshape(
        _NSPLIT, _NSTEP, _BLK).transpose(1, 0, 2)
    qrows = _B // _NSPLIT // _BLK
    in_specs = [pl.BlockSpec((1, _NSPLIT, _BLK), lambda i: (i, 0, 0))]
    for q in range(_NSPLIT):
        in_specs.append(
            pl.BlockSpec((_BLK, _C), lambda i, q=q: (q * qrows + i, 0)))
    out = pl.pallas_call(
        _loss_body,
        grid=(_NSTEP,),
        in_specs=in_specs,
        out_specs=pl.BlockSpec(memory_space=pltpu.SMEM),
        out_shape=jax.ShapeDtypeStruct((1, 1), jnp.float32),
        scratch_shapes=[pltpu.SMEM((1, 1), jnp.float32)],
    )(lab3, *([pred] * _NSPLIT))
    return out[0, 0]
